# Initial kernel scaffold; baseline (speedup 1.0000x reference)
import jax
import jax.numpy as jnp
from jax.experimental import pallas as pl


def kernel(x):
    raise NotImplementedError("write your pallas kernel here")


# speedup vs baseline: 9.0454x; 9.0454x over previous
import functools

import jax
import jax.numpy as jnp
from jax import lax
from jax.experimental import pallas as pl
from jax.experimental.pallas import tpu as pltpu
from jax.experimental.pallas import tpu_sc as plsc

B, C, N = 8, 3, 32768
K = 1024
NC, NS, L = 2, 16, 16
GPB = 4
NCHUNK = N // GPB
NSLICE = NCHUNK // L
BPS = NS // GPB
U = 4
BIGF = 3.0e38


def _fps_kernel(x_hbm, out_hbm, xv, yv, zv, dv, out_v, comm_v, row_v,
                p0_v, comm_sh):
    c = lax.axis_index("c")
    s = lax.axis_index("s")
    b = c * BPS + s // GPB
    lb = s // GPB
    g = s % GPB
    base = g * NCHUNK

    iota_i = lax.iota(jnp.int32, L)
    iota_f = iota_i.astype(jnp.float32)
    zeros_i = jnp.zeros((L,), jnp.int32)

    def fulli(v):
        return jnp.full((L,), v, jnp.int32)

    def fullf(v):
        return jnp.full((L,), v, jnp.float32)

    xoff = b * (C * N)
    pltpu.sync_copy(x_hbm.at[pl.ds(xoff + base, NCHUNK)], xv)
    pltpu.sync_copy(x_hbm.at[pl.ds(xoff + N + base, NCHUNK)], yv)
    pltpu.sync_copy(x_hbm.at[pl.ds(xoff + 2 * N + base, NCHUNK)], zv)
    pltpu.sync_copy(x_hbm.at[pl.ds(xoff, L)], p0_v.at[pl.ds(8, L)])
    pltpu.sync_copy(x_hbm.at[pl.ds(xoff + N, L)], p0_v.at[pl.ds(8 + L, L)])
    pltpu.sync_copy(x_hbm.at[pl.ds(xoff + 2 * N, L)],
                    p0_v.at[pl.ds(8 + 2 * L, L)])

    px0 = plsc.load_gather(p0_v, [fulli(8)])
    py0 = plsc.load_gather(p0_v, [fulli(8 + L)])
    pz0 = plsc.load_gather(p0_v, [fulli(8 + 2 * L)])

    inf_vec = fullf(jnp.inf)

    def init_body(t, _):
        dv[pl.ds(t * L, L)] = inf_vec
        return 0

    lax.fori_loop(0, NSLICE, init_body, 0)

    row_sel = jnp.minimum(iota_i, C - 1) * K
    out_mask0 = (iota_i < C) & (g == 0)
    coords0 = jnp.where(iota_i == 0, px0, jnp.where(iota_i == 1, py0, pz0))
    plsc.store_scatter(out_v, [row_sel], coords0, mask=out_mask0)

    nsteps = NSLICE // U

    def step(i, carry):
        px, py, pz = carry

        def inner(t, acc):
            accs = []
            for u in range(U):
                vmax, vidx = acc[2 * u], acc[2 * u + 1]
                t0 = (t * U + u) * L
                sl = pl.ds(t0, L)
                dx = xv[sl] - px
                dy = yv[sl] - py
                dz = zv[sl] - pz
                d = (dx * dx + dy * dy) + dz * dz
                dn = jnp.minimum(dv[sl], d)
                dv[sl] = dn
                better = dn > vmax
                tvec = lax.convert_element_type(t0, jnp.float32) + iota_f
                vmax = jnp.where(better, dn, vmax)
                vidx = jnp.where(better, tvec, vidx)
                accs += [vmax, vidx]
            return tuple(accs)

        acc0 = ()
        for _ in range(U):
            acc0 += (fullf(-jnp.inf), fullf(0.0))
        acc = lax.fori_loop(0, nsteps, inner, acc0)

        vmax, vidx = acc[0], acc[1]
        for u in range(1, U):
            vb, ib = acc[2 * u], acc[2 * u + 1]
            take = (vb > vmax) | ((vb == vmax) & (ib < vidx))
            vmax = jnp.where(take, vb, vmax)
            vidx = jnp.where(take, ib, vidx)

        m = jnp.max(vmax)
        li = jnp.min(jnp.where(vmax == m, vidx, BIGF))
        gi = li + lax.convert_element_type(base, jnp.float32)

        li_vec = jnp.full((L,), li).astype(jnp.int32)
        cx = plsc.load_gather(xv, [li_vec])
        cy = plsc.load_gather(yv, [li_vec])
        cz = plsc.load_gather(zv, [li_vec])

        row = jnp.where(
            iota_i == 0, jnp.full((L,), m),
            jnp.where(
                iota_i == 1, jnp.full((L,), gi),
                jnp.where(iota_i == 2, cx,
                          jnp.where(iota_i == 3, cy, cz))))
        row_v[...] = row
        boff = lax.rem(i, 2) * (BPS * GPB * L) + lb * (GPB * L)
        pltpu.sync_copy(row_v, comm_sh.at[pl.ds(boff + g * L, L)])
        plsc.subcore_barrier()
        pltpu.sync_copy(comm_sh.at[pl.ds(boff, GPB * L)], comm_v)

        rsel = jnp.minimum(iota_i, GPB - 1) * L
        vals = plsc.load_gather(comm_v, [rsel])
        gidx = plsc.load_gather(comm_v, [rsel + 1])
        lane_ok = iota_i < GPB
        m2 = jnp.max(jnp.where(lane_ok, vals, -jnp.inf))
        gwin = jnp.min(jnp.where(lane_ok & (vals == m2), gidx, BIGF))
        w_vec = (jnp.full((L,), gwin).astype(jnp.int32) // NCHUNK) * L
        nx = plsc.load_gather(comm_v, [w_vec + 2])
        ny = plsc.load_gather(comm_v, [w_vec + 3])
        nz = plsc.load_gather(comm_v, [w_vec + 4])

        coords = jnp.where(iota_i == 0, nx, jnp.where(iota_i == 1, ny, nz))
        plsc.store_scatter(out_v, [row_sel + i], coords, mask=out_mask0)
        return (nx, ny, nz)

    lax.fori_loop(1, K, step, (px0, py0, pz0))

    @pl.when(g == 0)
    def _():
        pltpu.sync_copy(out_v, out_hbm.at[pl.ds(b * (C * K), C * K)])


@jax.jit
def kernel(x):
    mesh = plsc.VectorSubcoreMesh(core_axis_name="c", subcore_axis_name="s",
                                  num_cores=NC, num_subcores=NS)
    run = pl.kernel(
        _fps_kernel,
        out_type=jax.ShapeDtypeStruct((B * C * K,), jnp.float32),
        mesh=mesh,
        compiler_params=pltpu.CompilerParams(needs_layout_passes=False),
        scratch_types=[
            pltpu.VMEM((NCHUNK,), jnp.float32),
            pltpu.VMEM((NCHUNK,), jnp.float32),
            pltpu.VMEM((NCHUNK,), jnp.float32),
            pltpu.VMEM((NCHUNK,), jnp.float32),
            pltpu.VMEM((C * K,), jnp.float32),
            pltpu.VMEM((GPB * L,), jnp.float32),
            pltpu.VMEM((L,), jnp.float32),
            pltpu.VMEM((8 + C * L + 8,), jnp.float32),
            pltpu.VMEM_SHARED((2 * BPS * GPB * L,), jnp.float32),
        ],
    )
    return run(x.reshape(B * C * N)).reshape(B, C, K)


# speedup vs baseline: 28.4400x; 3.1441x over previous
import functools

import jax
import jax.numpy as jnp
from jax import lax
from jax.experimental import pallas as pl
from jax.experimental.pallas import tpu as pltpu
from jax.experimental.pallas import tpu_sc as plsc

B, C, N = 8, 3, 32768
K = 1024
NC, NS, L = 2, 16, 16
GPB = 4
NCHUNK = N // GPB
NSLICE = NCHUNK // L
BPS = NS // GPB
U = 4
BIGF = 3.0e38


def _fps_kernel(x_hbm, out_hbm, xv, yv, zv, dv, out_v, comm_v, row_v,
                p0_v, comm_sh):
    c = lax.axis_index("c")
    s = lax.axis_index("s")
    b = c * BPS + s // GPB
    lb = s // GPB
    g = s % GPB
    base = g * NCHUNK

    iota_i = lax.iota(jnp.int32, L)
    iota_f = iota_i.astype(jnp.float32)
    zeros_i = jnp.zeros((L,), jnp.int32)

    def fulli(v):
        return jnp.full((L,), v, jnp.int32)

    def fullf(v):
        return jnp.full((L,), v, jnp.float32)

    xoff = b * (C * N)
    pltpu.sync_copy(x_hbm.at[pl.ds(xoff + base, NCHUNK)], xv)
    pltpu.sync_copy(x_hbm.at[pl.ds(xoff + N + base, NCHUNK)], yv)
    pltpu.sync_copy(x_hbm.at[pl.ds(xoff + 2 * N + base, NCHUNK)], zv)
    pltpu.sync_copy(x_hbm.at[pl.ds(xoff, L)], p0_v.at[pl.ds(8, L)])
    pltpu.sync_copy(x_hbm.at[pl.ds(xoff + N, L)], p0_v.at[pl.ds(8 + L, L)])
    pltpu.sync_copy(x_hbm.at[pl.ds(xoff + 2 * N, L)],
                    p0_v.at[pl.ds(8 + 2 * L, L)])

    px0 = plsc.load_gather(p0_v, [fulli(8)])
    py0 = plsc.load_gather(p0_v, [fulli(8 + L)])
    pz0 = plsc.load_gather(p0_v, [fulli(8 + 2 * L)])

    inf_vec = fullf(jnp.inf)

    def init_body(t, _):
        dv[pl.ds(t * L, L)] = inf_vec
        return 0

    lax.fori_loop(0, NSLICE, init_body, 0)

    row_sel = jnp.minimum(iota_i, C - 1) * K
    out_mask0 = (iota_i < C) & (g == 0)
    coords0 = jnp.where(iota_i == 0, px0, jnp.where(iota_i == 1, py0, pz0))
    plsc.store_scatter(out_v, [row_sel], coords0, mask=out_mask0)

    nsteps = NSLICE // U

    def step(i, carry):
        px, py, pz = carry

        def dist_body(t, a):
            vmax, vidx, tvec = a
            sl = pl.ds(t * L, L)
            dx = xv[sl] - px
            dy = yv[sl] - py
            dz = zv[sl] - pz
            d = (dx * dx + dy * dy) + dz * dz
            dn = jnp.minimum(dv[sl], d)
            dv[sl] = dn
            better = dn > vmax
            vmax = jnp.where(better, dn, vmax)
            vidx = jnp.where(better, tvec, vidx)
            return (vmax, vidx, tvec + jnp.float32(L))

        vmax, vidx, _ = plsc.parallel_loop(
            0, NSLICE, unroll=U,
            carry=(fullf(-jnp.inf), fullf(0.0), iota_f))(dist_body)

        m = jnp.max(vmax)
        li = jnp.min(jnp.where(vmax == m, vidx, BIGF))
        gi = li + lax.convert_element_type(base, jnp.float32)

        li_vec = jnp.full((L,), li).astype(jnp.int32)
        cx = plsc.load_gather(xv, [li_vec])
        cy = plsc.load_gather(yv, [li_vec])
        cz = plsc.load_gather(zv, [li_vec])

        row = jnp.where(
            iota_i == 0, jnp.full((L,), m),
            jnp.where(
                iota_i == 1, jnp.full((L,), gi),
                jnp.where(iota_i == 2, cx,
                          jnp.where(iota_i == 3, cy, cz))))
        row_v[...] = row
        boff = lax.rem(i, 2) * (BPS * GPB * L) + lb * (GPB * L)
        pltpu.sync_copy(row_v, comm_sh.at[pl.ds(boff + g * L, L)])
        plsc.subcore_barrier()
        pltpu.sync_copy(comm_sh.at[pl.ds(boff, GPB * L)], comm_v)

        rsel = jnp.minimum(iota_i, GPB - 1) * L
        vals = plsc.load_gather(comm_v, [rsel])
        gidx = plsc.load_gather(comm_v, [rsel + 1])
        lane_ok = iota_i < GPB
        m2 = jnp.max(jnp.where(lane_ok, vals, -jnp.inf))
        gwin = jnp.min(jnp.where(lane_ok & (vals == m2), gidx, BIGF))
        w_vec = (jnp.full((L,), gwin).astype(jnp.int32) // NCHUNK) * L
        nx = plsc.load_gather(comm_v, [w_vec + 2])
        ny = plsc.load_gather(comm_v, [w_vec + 3])
        nz = plsc.load_gather(comm_v, [w_vec + 4])

        coords = jnp.where(iota_i == 0, nx, jnp.where(iota_i == 1, ny, nz))
        plsc.store_scatter(out_v, [row_sel + i], coords, mask=out_mask0)
        return (nx, ny, nz)

    lax.fori_loop(1, K, step, (px0, py0, pz0))

    @pl.when(g == 0)
    def _():
        pltpu.sync_copy(out_v, out_hbm.at[pl.ds(b * (C * K), C * K)])


@jax.jit
def kernel(x):
    mesh = plsc.VectorSubcoreMesh(core_axis_name="c", subcore_axis_name="s",
                                  num_cores=NC, num_subcores=NS)
    run = pl.kernel(
        _fps_kernel,
        out_type=jax.ShapeDtypeStruct((B * C * K,), jnp.float32),
        mesh=mesh,
        compiler_params=pltpu.CompilerParams(needs_layout_passes=False),
        scratch_types=[
            pltpu.VMEM((NCHUNK,), jnp.float32),
            pltpu.VMEM((NCHUNK,), jnp.float32),
            pltpu.VMEM((NCHUNK,), jnp.float32),
            pltpu.VMEM((NCHUNK,), jnp.float32),
            pltpu.VMEM((C * K,), jnp.float32),
            pltpu.VMEM((GPB * L,), jnp.float32),
            pltpu.VMEM((L,), jnp.float32),
            pltpu.VMEM((8 + C * L + 8,), jnp.float32),
            pltpu.VMEM_SHARED((2 * BPS * GPB * L,), jnp.float32),
        ],
    )
    return run(x.reshape(B * C * N)).reshape(B, C, K)


# speedup vs baseline: 28.4400x; 1.0000x over previous
import jax
import jax.numpy as jnp
from jax import lax
from jax.experimental import pallas as pl
from jax.experimental.pallas import tpu as pltpu
from jax.experimental.pallas import tpu_sc as plsc

B, C, N = 8, 3, 32768
K = 1024
NC, NS, L = 2, 16, 16
GPB = 4
NCHUNK = N // GPB
NSLICE = NCHUNK // L
BPS = NS // GPB
U = 4
BIGF = 3.0e38


def _fps_kernel(x_hbm, out_hbm, xv, yv, zv, dv, out_v, comm_v, row_v,
                p0_v, comm_sh):
    c = lax.axis_index("c")
    s = lax.axis_index("s")
    b = c * BPS + s // GPB
    lb = s // GPB
    g = s % GPB
    base = g * NCHUNK

    iota_i = lax.iota(jnp.int32, L)
    iota_f = iota_i.astype(jnp.float32)

    def fulli(v):
        return jnp.full((L,), v, jnp.int32)

    def fullf(v):
        return jnp.full((L,), v, jnp.float32)

    xoff = b * (C * N)
    pltpu.sync_copy(x_hbm.at[pl.ds(xoff + base, NCHUNK)], xv)
    pltpu.sync_copy(x_hbm.at[pl.ds(xoff + N + base, NCHUNK)], yv)
    pltpu.sync_copy(x_hbm.at[pl.ds(xoff + 2 * N + base, NCHUNK)], zv)
    pltpu.sync_copy(x_hbm.at[pl.ds(xoff, L)], p0_v.at[pl.ds(8, L)])
    pltpu.sync_copy(x_hbm.at[pl.ds(xoff + N, L)], p0_v.at[pl.ds(8 + L, L)])
    pltpu.sync_copy(x_hbm.at[pl.ds(xoff + 2 * N, L)],
                    p0_v.at[pl.ds(8 + 2 * L, L)])

    px0 = plsc.load_gather(p0_v, [fulli(8)])
    py0 = plsc.load_gather(p0_v, [fulli(8 + L)])
    pz0 = plsc.load_gather(p0_v, [fulli(8 + 2 * L)])

    inf_vec = fullf(jnp.inf)

    def init_body(t, _):
        dv[pl.ds(t * L, L)] = inf_vec
        return 0

    lax.fori_loop(0, NSLICE, init_body, 0)

    row_sel = jnp.minimum(iota_i, C - 1) * K
    out_mask0 = (iota_i < C) & (g == 0)
    coords0 = jnp.where(iota_i == 0, px0, jnp.where(iota_i == 1, py0, pz0))
    plsc.store_scatter(out_v, [row_sel], coords0, mask=out_mask0)

    def step(i, carry):
        px, py, pz = carry

        def dist_body(t, a):
            vmax, vidx, tvec = a
            sl = pl.ds(t * L, L)
            dx = xv[sl] - px
            dy = yv[sl] - py
            dz = zv[sl] - pz
            d = (dx * dx + dy * dy) + dz * dz
            dn = jnp.minimum(dv[sl], d)
            dv[sl] = dn
            better = dn > vmax
            vmax = jnp.where(better, dn, vmax)
            vidx = jnp.where(better, tvec, vidx)
            return (vmax, vidx, tvec + jnp.float32(L))

        vmax, vidx, _ = plsc.parallel_loop(
            0, NSLICE, unroll=U,
            carry=(fullf(-jnp.inf), fullf(0.0), iota_f))(dist_body)

        m = jnp.max(vmax)
        li = jnp.min(jnp.where(vmax == m, vidx, BIGF))
        gi = li + lax.convert_element_type(base, jnp.float32)

        li_vec = jnp.full((L,), li).astype(jnp.int32)
        cx = plsc.load_gather(xv, [li_vec])
        cy = plsc.load_gather(yv, [li_vec])
        cz = plsc.load_gather(zv, [li_vec])

        row = jnp.where(
            iota_i == 0, jnp.full((L,), m),
            jnp.where(
                iota_i == 1, jnp.full((L,), gi),
                jnp.where(iota_i == 2, cx,
                          jnp.where(iota_i == 3, cy, cz))))
        row_v[...] = row
        boff = lax.rem(i, 2) * (BPS * GPB * L) + lb * (GPB * L)
        pltpu.sync_copy(row_v, comm_sh.at[pl.ds(boff + g * L, L)])
        plsc.subcore_barrier()
        pltpu.sync_copy(comm_sh.at[pl.ds(boff, GPB * L)], comm_v)

        rsel = jnp.minimum(iota_i, GPB - 1) * L
        vals = plsc.load_gather(comm_v, [rsel])
        gidx = plsc.load_gather(comm_v, [rsel + 1])
        lane_ok = iota_i < GPB
        m2 = jnp.max(jnp.where(lane_ok, vals, -jnp.inf))
        gwin = jnp.min(jnp.where(lane_ok & (vals == m2), gidx, BIGF))
        w_vec = (jnp.full((L,), gwin).astype(jnp.int32) // NCHUNK) * L
        nx = plsc.load_gather(comm_v, [w_vec + 2])
        ny = plsc.load_gather(comm_v, [w_vec + 3])
        nz = plsc.load_gather(comm_v, [w_vec + 4])

        coords = jnp.where(iota_i == 0, nx, jnp.where(iota_i == 1, ny, nz))
        plsc.store_scatter(out_v, [row_sel + i], coords, mask=out_mask0)
        return (nx, ny, nz)

    lax.fori_loop(1, K, step, (px0, py0, pz0))

    @pl.when(g == 0)
    def _():
        pltpu.sync_copy(out_v, out_hbm.at[pl.ds(b * (C * K), C * K)])


@jax.jit
def kernel(x):
    mesh = plsc.VectorSubcoreMesh(core_axis_name="c", subcore_axis_name="s",
                                  num_cores=NC, num_subcores=NS)
    run = pl.kernel(
        _fps_kernel,
        out_type=jax.ShapeDtypeStruct((B * C * K,), jnp.float32),
        mesh=mesh,
        compiler_params=pltpu.CompilerParams(needs_layout_passes=False),
        scratch_types=[
            pltpu.VMEM((NCHUNK,), jnp.float32),
            pltpu.VMEM((NCHUNK,), jnp.float32),
            pltpu.VMEM((NCHUNK,), jnp.float32),
            pltpu.VMEM((NCHUNK,), jnp.float32),
            pltpu.VMEM((C * K,), jnp.float32),
            pltpu.VMEM((GPB * L,), jnp.float32),
            pltpu.VMEM((L,), jnp.float32),
            pltpu.VMEM((8 + C * L + 8,), jnp.float32),
            pltpu.VMEM_SHARED((2 * BPS * GPB * L,), jnp.float32),
        ],
    )
    return run(x.reshape(B * C * N)).reshape(B, C, K)
